# baseline (speedup 1.0000x reference)
import functools

import jax
import jax.numpy as jnp
from jax import lax
from jax.experimental import pallas as pl
from jax.experimental.pallas import tpu as pltpu
from jax.experimental.pallas import tpu_sc as plsc

_B, _L, _D = 4096, 200, 32
_N = _B * _L
_CHUNK = 1280
_V = 1000000


def _build_gather():
    info = plsc.get_sparse_core_info()
    nc = info.num_cores
    nw = nc * info.num_subcores
    n_per_w = _N // nw
    nblk = n_per_w // _CHUNK
    mesh = plsc.VectorSubcoreMesh(core_axis_name="c", subcore_axis_name="s")

    @functools.partial(
        pl.kernel,
        mesh=mesh,
        out_type=jax.ShapeDtypeStruct((_N, _D), jnp.float32),
        compiler_params=pltpu.CompilerParams(use_tc_tiling_on_sc=False),
        scratch_types=[
            pltpu.VMEM((n_per_w,), jnp.int32),
            pltpu.VMEM((_CHUNK, _D), jnp.float32),
            pltpu.VMEM((_CHUNK, _D), jnp.float32),
            pltpu.SemaphoreType.DMA,
            pltpu.SemaphoreType.DMA,
        ],
    )
    def gather(idx_hbm, table_hbm, out_hbm, idx_v, rows0, rows1, sem0, sem1):
        wid = lax.axis_index("s") * nc + lax.axis_index("c")
        base0 = wid * n_per_w

        pltpu.sync_copy(idx_hbm.at[pl.ds(base0, n_per_w)], idx_v)

        def fire(blk, rows_v, sem):
            pltpu.async_copy(
                table_hbm.at[idx_v.at[pl.ds(blk * _CHUNK, _CHUNK)]],
                rows_v,
                sem,
            )

        def drain(rows_v, sem):
            pltpu.make_async_copy(out_hbm.at[pl.ds(0, _CHUNK)], rows_v, sem).wait()

        def writeback(blk, rows_v):
            pltpu.sync_copy(rows_v, out_hbm.at[pl.ds(base0 + blk * _CHUNK, _CHUNK)])

        fire(0, rows0, sem0)

        def body(g2, carry):
            g = g2 * 2
            fire(g + 1, rows1, sem1)
            drain(rows0, sem0)
            writeback(g, rows0)

            @pl.when(g + 2 < nblk)
            def _():
                fire(g + 2, rows0, sem0)

            drain(rows1, sem1)
            writeback(g + 1, rows1)
            return carry

        lax.fori_loop(0, nblk // 2, body, 0)

    return gather


def _build_pack():
    info = plsc.get_sparse_core_info()
    nc = info.num_cores
    nw = nc * info.num_subcores
    n_items = _L * (_B // 128)
    items_per_w = n_items // nw
    mesh = plsc.VectorSubcoreMesh(core_axis_name="c", subcore_axis_name="s")

    @functools.partial(
        pl.kernel,
        mesh=mesh,
        out_type=jax.ShapeDtypeStruct((_L, _D, _B), jnp.float32),
        compiler_params=pltpu.CompilerParams(needs_layout_passes=False),
        scratch_types=[
            pltpu.VMEM((_D, 128), jnp.float32),
            pltpu.VMEM((_D, 128), jnp.float32),
        ],
    )
    def pack(rows_hbm, out_hbm, in_v, out_v):
        wid = lax.axis_index("s") * nc + lax.axis_index("c")

        def body(i, carry):
            item = wid * items_per_w + i
            l = item // 32
            tb = item % 32
            iota = lax.iota(jnp.int32, 16)
            colpats = [(iota % 4) * _D + e for e in range(_D)]
            rowpats = [(iota // 4) + g * 4 for g in range(8)]
            pltpu.sync_copy(rows_hbm.at[pl.ds(l * 1024 + tb * _D, _D)], in_v)
            for g in range(8):
                for e0 in range(0, _D, 8):
                    vals = [
                        plsc.load_gather(in_v, [rowpats[g], colpats[e0 + k]])
                        for k in range(8)
                    ]
                    for k in range(8):
                        out_v[e0 + k, pl.ds(g * 16, 16)] = vals[k]
            pltpu.sync_copy(out_v, out_hbm.at[l, :, pl.ds(tb * 128, 128)])
            return carry

        lax.fori_loop(0, items_per_w, body, 0)

    return pack


def _tc_pack_body(i_ref, o_ref):
    o_ref[0] = i_ref[:].reshape(128, _D).T


def _build_pack_tc():
    return pl.pallas_call(
        _tc_pack_body,
        grid=(_L, _B // 128),
        in_specs=[pl.BlockSpec((_D, 128), lambda l, t: (l * 32 + t, 0))],
        out_specs=pl.BlockSpec((1, _D, 128), lambda l, t: (l, 0, t)),
        out_shape=jax.ShapeDtypeStruct((_L, _D, _B), jnp.float32),
    )


_gather = _build_gather()
_pack = _build_pack_tc()


def kernel(x, table):
    idx_t = x.T.reshape(_N)
    rows = _gather(idx_t, table)
    out3 = _pack(rows.reshape(_N * _D // 128, 128))
    return out3.transpose(2, 0, 1)


# speedup vs baseline: 3.7103x; 3.7103x over previous
import functools

import jax
import jax.numpy as jnp
from jax import lax
from jax.experimental import pallas as pl
from jax.experimental.pallas import tpu as pltpu
from jax.experimental.pallas import tpu_sc as plsc

_B, _L, _D = 4096, 200, 32
_N = _B * _L
_IW = 128
_KD = 10
_CHUNK = _IW * _KD


def _build():
    info = plsc.get_sparse_core_info()
    nc = info.num_cores
    nw = nc * info.num_subcores
    n_per_w = _N // nw
    nblk = n_per_w // _CHUNK
    rows_per_w = n_per_w // _IW
    mesh = plsc.VectorSubcoreMesh(core_axis_name="c", subcore_axis_name="s")

    @functools.partial(
        pl.kernel,
        mesh=mesh,
        out_type=jax.ShapeDtypeStruct((_N, _D), jnp.float32),
        compiler_params=pltpu.CompilerParams(use_tc_tiling_on_sc=False),
        scratch_types=[
            pltpu.VMEM((rows_per_w, _IW), jnp.int32),
            pltpu.VMEM((_CHUNK, _D), jnp.float32),
            pltpu.VMEM((_CHUNK, _D), jnp.float32),
            pltpu.SemaphoreType.DMA,
            pltpu.SemaphoreType.DMA,
        ],
    )
    def gather(idx_hbm, table_hbm, out_hbm, idx_v, rows0, rows1, sem0, sem1):
        wid = lax.axis_index("s") * nc + lax.axis_index("c")
        row0 = wid * rows_per_w
        base0 = wid * n_per_w

        pltpu.sync_copy(idx_hbm.at[pl.ds(row0, rows_per_w)], idx_v)

        def fire(blk, rows_v, sem):
            for j in range(_KD):
                pltpu.async_copy(
                    table_hbm.at[idx_v.at[blk * _KD + j]],
                    rows_v.at[pl.ds(j * _IW, _IW)],
                    sem,
                )

        def drain(rows_v, sem):
            pltpu.make_async_copy(out_hbm.at[pl.ds(0, _CHUNK)], rows_v, sem).wait()

        def writeback(blk, rows_v):
            pltpu.sync_copy(rows_v, out_hbm.at[pl.ds(base0 + blk * _CHUNK, _CHUNK)])

        fire(0, rows0, sem0)

        def body(g2, carry):
            g = g2 * 2
            fire(g + 1, rows1, sem1)
            drain(rows0, sem0)
            writeback(g, rows0)

            @pl.when(g + 2 < nblk)
            def _():
                fire(g + 2, rows0, sem0)

            drain(rows1, sem1)
            writeback(g + 1, rows1)
            return carry

        lax.fori_loop(0, nblk // 2, body, 0)

    return gather


def _build_pack():
    info = plsc.get_sparse_core_info()
    nc = info.num_cores
    nw = nc * info.num_subcores
    n_items = _L * (_B // 128)
    groups_per_w = n_items // nw // 4
    mesh = plsc.VectorSubcoreMesh(core_axis_name="c", subcore_axis_name="s")

    @functools.partial(
        pl.kernel,
        mesh=mesh,
        out_type=jax.ShapeDtypeStruct((_L, _D, _B), jnp.float32),
        compiler_params=pltpu.CompilerParams(needs_layout_passes=False),
        scratch_types=[
            pltpu.VMEM((128, 128), jnp.float32),
            pltpu.VMEM((_D, 512), jnp.float32),
        ],
    )
    def pack(rows_hbm, out_hbm, in_v, out_v):
        wid = lax.axis_index("s") * nc + lax.axis_index("c")

        def body(i, carry):
            item = wid * (groups_per_w * 4) + i * 4
            l = item // 32
            tb = item % 32
            iota = lax.iota(jnp.int32, 16)
            colpats = [(iota % 4) * _D + e for e in range(_D)]
            rowpats = [(iota // 4) + g * 4 for g in range(32)]
            pltpu.sync_copy(rows_hbm.at[pl.ds(l * 1024 + tb * _D, 128)], in_v)
            for g in range(32):
                for e0 in range(0, _D, 8):
                    vals = [
                        plsc.load_gather(in_v, [rowpats[g], colpats[e0 + k]])
                        for k in range(8)
                    ]
                    for k in range(8):
                        out_v[e0 + k, pl.ds(g * 16, 16)] = vals[k]
            pltpu.sync_copy(out_v, out_hbm.at[l, :, pl.ds(tb * 128, 512)])
            return carry

        lax.fori_loop(0, groups_per_w, body, 0)

    return pack


_gather = _build()
_pack = _build_pack()


def kernel(x, table):
    idx_t = x.T.reshape(_N // _IW, _IW)
    rows = _gather(idx_t, table)
    out3 = _pack(rows.reshape(_N * _D // 128, 128))
    return out3.transpose(2, 0, 1)
